# baseline (speedup 1.0000x reference)
import jax, jax.numpy as jnp
from jax import lax

def _bf(x):
    return x.astype(jnp.bfloat16).astype(jnp.float32)


def kernel(ego_psm, cav_psm, W1, b1, W2, b2):
    single = jnp.concatenate([ego_psm, cav_psm], axis=0)[None, ...]
    f = lax.conv_general_dilated(_bf(single), _bf(W1), (1, 1), ((2, 2), (2, 2)),
        dimension_numbers=("NCHW", "OIHW", "NCHW"), precision=lax.Precision.HIGHEST)
    f = f + b1[None, :, None, None]
    f = lax.conv_general_dilated(_bf(f), _bf(W2), (1, 1), ((0, 0), (0, 0)),
        dimension_numbers=("NCHW", "OIHW", "NCHW"), precision=lax.Precision.HIGHEST)
    f = (f + b2[None, :, None, None])[0]
    k = f.size // 2
    tv, _ = lax.top_k(f.reshape(-1), k)
    return (f >= tv.min()).astype(jnp.float32)


# speedup vs baseline: 6.5919x; 6.5919x over previous
import jax
import jax.numpy as jnp
from jax import lax
from jax.experimental import pallas as pl
from jax.experimental.pallas import tpu as pltpu

_H = 384
_W = 384
_C = 4
_KS = 5
_OC = 192
_DYSTRIDE = 32
_KDIM = _KS * _DYSTRIDE
_RB = 8
_NB = _H // _RB
_KSEL = (_H * _W) // 2


def _monotone_u32(f):
    u = lax.bitcast_convert_type(f, jnp.uint32)
    sign = (u >> jnp.uint32(31)) != jnp.uint32(0)
    return jnp.where(sign, u ^ jnp.uint32(0xFFFFFFFF), u | jnp.uint32(0x80000000))


def _body(xpad_ref, w1_ref, b1_ref, w2_ref, b2_ref, out_ref, patch_ref, f_ref):
    patch_ref[...] = jnp.zeros((_KDIM, _W), jnp.bfloat16)
    for dyp in range(4):
        for c in range(_C):
            row = xpad_ref[dyp, c, :].reshape(1, _W + 4)
            for dx in range(_KS):
                base = (dyp + 1) * _DYSTRIDE + c * _KS + dx
                patch_ref[base:base + 1, :] = row[:, dx:dx + _W]

    w1 = w1_ref[...]
    b1 = b1_ref[...]
    w2 = w2_ref[...].astype(jnp.float32)
    b2v = jnp.sum(b2_ref[...])

    def block_step(blk, _):
        for j in range(_RB):
            patch_ref[0:_KDIM - _DYSTRIDE, :] = patch_ref[_DYSTRIDE:_KDIM, :]
            for c in range(_C):
                row = xpad_ref[pl.ds(blk * _RB + j + 4, 1), c, :]
                for dx in range(_KS):
                    base = 4 * _DYSTRIDE + c * _KS + dx
                    patch_ref[base:base + 1, :] = row[:, dx:dx + _W]
            f1 = lax.dot_general(w1, patch_ref[...], (((1,), (0,)), ((), ())),
                                 preferred_element_type=jnp.float32)
            f1 = f1 + b1
            f1b = f1.astype(jnp.bfloat16).astype(jnp.float32)
            f2 = jnp.sum(f1b * w2, axis=0, keepdims=True) + b2v
            f_ref[pl.ds(blk, 1), j, :] = f2
        return 0

    lax.fori_loop(0, _NB, block_step, 0)

    keys = _monotone_u32(f_ref[...])

    def search_step(i, prefix):
        bit = jnp.uint32(31) - i.astype(jnp.uint32)
        cand = prefix | (jnp.uint32(1) << bit)
        cnt = jnp.sum((keys >= cand).astype(jnp.int32))
        return jnp.where(cnt >= _KSEL, cand, prefix)

    thr = lax.fori_loop(0, 32, search_step, jnp.uint32(0))
    mask = (keys >= thr).astype(jnp.float32)
    for blk in range(_NB):
        out_ref[0, blk * _RB:(blk + 1) * _RB, :] = mask[blk]


def kernel(ego_psm, cav_psm, W1, b1, W2, b2):
    x = jnp.concatenate([ego_psm, cav_psm], axis=0)
    xpad = jnp.pad(x, ((0, 0), (2, 2), (2, 2))).astype(jnp.bfloat16)
    xpad_t = jnp.transpose(xpad, (1, 0, 2))
    w1p = jnp.concatenate(
        [jnp.pad(W1[:, :, dy, :].reshape(_OC, _C * _KS), ((0, 0), (0, _DYSTRIDE - _C * _KS)))
         for dy in range(_KS)], axis=1).astype(jnp.bfloat16)
    b1r = b1.reshape(_OC, 1)
    w2r = W2.reshape(_OC, 1).astype(jnp.bfloat16)
    b2r = b2.reshape(1, 1)
    return pl.pallas_call(
        _body,
        out_shape=jax.ShapeDtypeStruct((1, _H, _W), jnp.float32),
        scratch_shapes=[
            pltpu.VMEM((_KDIM, _W), jnp.bfloat16),
            pltpu.VMEM((_NB, _RB, _W), jnp.float32),
        ],
    )(xpad_t, w1p, b1r, w2r, b2r)
